# Initial kernel scaffold; baseline (speedup 1.0000x reference)
import functools
import math

import jax
import jax.numpy as jnp
from jax import lax
from jax.experimental import pallas as pl
from jax.experimental.pallas import tpu as pltpu
from jax.experimental.pallas import tpu_sc as plsc

NV = 50000
NS = 1600000
DT = 0.001
DASHPOT = 0.1
DRAG = 0.1
GRAV_Z = -9.8
DAMP = math.exp(-DT * DRAG)

NC, NSC, L = 2, 16, 16
NT = NC * NSC
NV_PAD = 50176
RPT = NV_PAD // NSC
RPT32 = NV_PAD // NT
PR = 784
SPT = NS // NT
CH = 2048
NCHUNK = 25
SPT_PAD = CH * NCHUNK
NS_PAD = SPT_PAD * NT
GPC = CH // L

_MESH = plsc.VectorSubcoreMesh(
    core_axis_name="c", subcore_axis_name="s", num_cores=NC, num_subcores=NSC)

f32 = jnp.float32
i32 = jnp.int32


def _rsqrt(x):
    xb = lax.bitcast_convert_type(x, i32)
    r = lax.bitcast_convert_type(jnp.int32(0x5F3759DF) - (xb >> 1), f32)
    xh = x * 0.5
    for _ in range(3):
        r = r * (1.5 - xh * r * r)
    return r


def _iota():
    return lax.broadcasted_iota(i32, (L,), 0)


def _col(c):
    return jnp.full((L,), c, i32)


def _integrate_group(g, PS, PF0, PF1, PM, CEb, CFb):
    iot = _iota()
    rvec = g * L + iot
    m = PM[pl.ds(g * L, L)]
    ce = CEb[...]
    cf = CFb[...]
    x = [plsc.load_gather(PS, [rvec, _col(c)]) for c in range(3)]
    v = [plsc.load_gather(PS, [rvec, _col(4 + c)]) for c in range(3)]
    f = [plsc.load_gather(PF0, [rvec, _col(c)]) +
         plsc.load_gather(PF1, [rvec, _col(c)]) for c in range(3)]
    f[2] = f[2] + GRAV_Z * m
    dtm = DT / m
    vn = [(v[c] + dtm * f[c]) * DAMP for c in range(3)]
    xn = [x[c] + DT * vn[c] for c in range(3)]
    vz = vn[2]
    mask = (xn[2] < 0.0) & (vz < 0.0)
    tao_sq = vn[0] * vn[0] + vn[1] * vn[1]
    ts_safe = jnp.where(tao_sq > 0.0, tao_sq, 1.0)
    inv_tao = _rsqrt(ts_safe)
    a = jnp.maximum(0.0, 1.0 - cf * (1.0 + ce) * jnp.abs(vz) * inv_tao)
    v3 = [jnp.where(mask, a * vn[0], vn[0]),
          jnp.where(mask, a * vn[1], vn[1]),
          jnp.where(mask, -ce * vz, vz)]
    x3 = [xn[0], xn[1], jnp.where(mask, 0.0, xn[2])]
    return rvec, x3, v3


def _make_edge(integrate, emit_sf):
    outs = [jax.ShapeDtypeStruct((NV_PAD, 8), f32),
            jax.ShapeDtypeStruct((NC, NV_PAD, 4), f32)]
    if emit_sf:
        outs.append(jax.ShapeDtypeStruct((NS_PAD, 3), f32))

    scratch = (
        pltpu.VMEM_SHARED((NV_PAD, 8), f32),
        pltpu.VMEM_SHARED((NV_PAD, 4), f32),
        pltpu.VMEM((PR, 8), f32),
        pltpu.VMEM((PR, 4), f32),
        pltpu.VMEM((PR, 4), f32),
        pltpu.VMEM((PR,), f32),
        pltpu.VMEM((PR, 8), f32),
        pltpu.VMEM((L,), f32),
        pltpu.VMEM((L,), f32),
        pltpu.VMEM((16, 128), i32),
        pltpu.VMEM((16, 128), i32),
        pltpu.VMEM((CH,), f32),
        pltpu.VMEM((CH,), f32),
        pltpu.VMEM((CH, 8), f32),
        pltpu.VMEM((CH, 8), f32),
        pltpu.VMEM((CH, 4), f32),
        pltpu.VMEM((CH, 4), f32),
        pltpu.VMEM((CH, 3), f32),
        pltpu.SemaphoreType.DMA,
        pltpu.SemaphoreType.DMA,
    )

    @functools.partial(pl.kernel, out_type=tuple(outs), mesh=_MESH,
                       scratch_types=scratch,
                       name=f"edge_i{int(integrate)}_sf{int(emit_sf)}")
    def call(S_in, F_in, masses, ce16, cf16, idx1v, idx2v, restv, yv, z4,
             *rest):
        if emit_sf:
            S_out, F_out, SF_out = rest[:3]
            scr = rest[3:]
        else:
            S_out, F_out = rest[:2]
            SF_out = None
            scr = rest[2:]
        (S_sh, F_sh, PS, PF0, PF1, PM, PSN, CEb, CFb, I1, I2, RB, YB,
         P1, P2, Q1, Q2, SFB, gsem, ssem) = scr
        cid = lax.axis_index("c")
        sid = lax.axis_index("s")
        wid = cid * NSC + sid
        tb = sid * RPT

        pltpu.sync_copy(z4.at[pl.ds(tb, RPT)], F_sh.at[pl.ds(tb, RPT)])
        pltpu.sync_copy(ce16, CEb)
        pltpu.sync_copy(cf16, CFb)

        if not integrate:
            pltpu.sync_copy(S_in.at[pl.ds(tb, RPT)], S_sh.at[pl.ds(tb, RPT)])
        else:
            for k in range(RPT // PR):
                rows0 = tb + k * PR
                pltpu.sync_copy(S_in.at[pl.ds(rows0, PR)], PS)
                pltpu.sync_copy(F_in.at[0, pl.ds(rows0, PR)], PF0)
                pltpu.sync_copy(F_in.at[1, pl.ds(rows0, PR)], PF1)
                pltpu.sync_copy(masses.at[pl.ds(rows0, PR)], PM)

                @pl.loop(0, PR // L)
                def _grp(g):
                    rvec, x3, v3 = _integrate_group(g, PS, PF0, PF1, PM,
                                                    CEb, CFb)
                    for c in range(3):
                        plsc.store_scatter(PSN, [rvec, _col(c)], x3[c])
                        plsc.store_scatter(PSN, [rvec, _col(4 + c)], v3[c])

                pltpu.sync_copy(PSN, S_sh.at[pl.ds(rows0, PR)])

                @pl.when(cid == 0)
                def _():
                    pltpu.sync_copy(PSN, S_out.at[pl.ds(rows0, PR)])

        plsc.subcore_barrier()

        @pl.loop(0, NCHUNK)
        def _chunk(ch):
            base = wid * SPT_PAD + ch * CH
            b128 = wid * (SPT_PAD // 128) + ch * 16
            pltpu.sync_copy(idx1v.at[pl.ds(b128, 16)], I1)
            pltpu.sync_copy(idx2v.at[pl.ds(b128, 16)], I2)
            pltpu.sync_copy(restv.at[pl.ds(base, CH)], RB)
            pltpu.sync_copy(yv.at[pl.ds(base, CH)], YB)
            gds = []
            for j in range(16):
                gds.append(pltpu.async_copy(
                    S_sh.at[I1.at[j]], P1.at[pl.ds(j * 128, 128)], gsem))
                gds.append(pltpu.async_copy(
                    S_sh.at[I2.at[j]], P2.at[pl.ds(j * 128, 128)], gsem))
            for d in gds:
                d.wait()

            @pl.loop(0, GPC)
            def _grp(g):
                iot = _iota()
                rvec = g * L + iot
                x1 = [plsc.load_gather(P1, [rvec, _col(c)]) for c in range(3)]
                v1 = [plsc.load_gather(P1, [rvec, _col(4 + c)])
                      for c in range(3)]
                x2 = [plsc.load_gather(P2, [rvec, _col(c)]) for c in range(3)]
                v2 = [plsc.load_gather(P2, [rvec, _col(4 + c)])
                      for c in range(3)]
                d0 = x2[0] - x1[0]
                d1 = x2[1] - x1[1]
                d2 = x2[2] - x1[2]
                dn2 = d0 * d0 + d1 * d1 + d2 * d2
                r = _rsqrt(dn2)
                dn = dn2 * r
                rl = RB[pl.ds(g * L, L)]
                e = jnp.exp(YB[pl.ds(g * L, L)])
                s = e * (dn / rl) - e
                vr = ((v2[0] - v1[0]) * d0 + (v2[1] - v1[1]) * d1 +
                      (v2[2] - v1[2]) * d2)
                ts = (s + DASHPOT * vr * r) * r
                fx, fy, fz = ts * d0, ts * d1, ts * d2
                plsc.store_scatter(Q1, [rvec, _col(0)], fx)
                plsc.store_scatter(Q1, [rvec, _col(1)], fy)
                plsc.store_scatter(Q1, [rvec, _col(2)], fz)
                plsc.store_scatter(Q2, [rvec, _col(0)], -fx)
                plsc.store_scatter(Q2, [rvec, _col(1)], -fy)
                plsc.store_scatter(Q2, [rvec, _col(2)], -fz)
                if emit_sf:
                    sr = s * r
                    plsc.store_scatter(SFB, [rvec, _col(0)], sr * d0)
                    plsc.store_scatter(SFB, [rvec, _col(1)], sr * d1)
                    plsc.store_scatter(SFB, [rvec, _col(2)], sr * d2)

            sds = []
            for j in range(16):
                sds.append(pltpu.async_copy(
                    Q1.at[pl.ds(j * 128, 128)], F_sh.at[I1.at[j]], ssem,
                    add=True))
                sds.append(pltpu.async_copy(
                    Q2.at[pl.ds(j * 128, 128)], F_sh.at[I2.at[j]], ssem,
                    add=True))
            if emit_sf:
                pltpu.sync_copy(SFB, SF_out.at[pl.ds(base, CH)])
            for d in sds:
                d.wait()

        plsc.subcore_barrier()
        pltpu.sync_copy(F_sh.at[pl.ds(tb, RPT)],
                        F_out.at[cid, pl.ds(tb, RPT)])

    return call


def _make_final():
    scratch = (
        pltpu.VMEM((PR, 8), f32),
        pltpu.VMEM((PR, 4), f32),
        pltpu.VMEM((PR, 4), f32),
        pltpu.VMEM((PR,), f32),
        pltpu.VMEM((PR, 3), f32),
        pltpu.VMEM((L,), f32),
        pltpu.VMEM((L,), f32),
    )

    @functools.partial(pl.kernel,
                       out_type=jax.ShapeDtypeStruct((NV_PAD, 3), f32),
                       mesh=_MESH, scratch_types=scratch,
                       name="final_integrate")
    def call(S_in, F_in, masses, ce16, cf16, x_out,
             PS, PF0, PF1, PM, XB, CEb, CFb):
        cid = lax.axis_index("c")
        sid = lax.axis_index("s")
        wid = cid * NSC + sid
        tb = wid * RPT32
        pltpu.sync_copy(ce16, CEb)
        pltpu.sync_copy(cf16, CFb)
        for k in range(RPT32 // PR):
            rows0 = tb + k * PR
            pltpu.sync_copy(S_in.at[pl.ds(rows0, PR)], PS)
            pltpu.sync_copy(F_in.at[0, pl.ds(rows0, PR)], PF0)
            pltpu.sync_copy(F_in.at[1, pl.ds(rows0, PR)], PF1)
            pltpu.sync_copy(masses.at[pl.ds(rows0, PR)], PM)

            @pl.loop(0, PR // L)
            def _grp(g):
                rvec, x3, _ = _integrate_group(g, PS, PF0, PF1, PM, CEb, CFb)
                for c in range(3):
                    plsc.store_scatter(XB, [rvec, _col(c)], x3[c])

            pltpu.sync_copy(XB, x_out.at[pl.ds(rows0, PR)])

    return call


_edge0 = _make_edge(integrate=False, emit_sf=False)
_edge_mid = _make_edge(integrate=True, emit_sf=False)
_edge_sf = _make_edge(integrate=True, emit_sf=True)
_final = _make_final()


def kernel(init_vertices, init_springs, init_rest_lengths, init_masses,
           spring_Y, collide_elas, collide_fric):
    S0 = jnp.zeros((NV_PAD, 8), f32).at[:NV, 0:3].set(init_vertices)
    masses_p = jnp.ones((NV_PAD,), f32).at[:NV].set(init_masses)
    idx1 = init_springs[:, 0]
    idx2 = init_springs[:, 1]

    def pad_springs(a, fill):
        a2 = a.reshape(NT, SPT)
        padblock = jnp.full((NT, SPT_PAD - SPT), fill, a.dtype)
        return jnp.concatenate([a2, padblock], axis=1).reshape(-1)

    i1p = pad_springs(idx1, NV).reshape(NS_PAD // 128, 128)
    i2p = pad_springs(idx2, NV).reshape(NS_PAD // 128, 128)
    restp = pad_springs(init_rest_lengths, 1.0)
    yp = pad_springs(spring_Y, 0.0)
    z4 = jnp.zeros((NV_PAD, 4), f32)
    ce16 = jnp.full((L,), collide_elas, f32)
    cf16 = jnp.full((L,), collide_fric, f32)
    fdummy = jnp.zeros((NC, NV_PAD, 4), f32)

    common = (masses_p, ce16, cf16, i1p, i2p, restp, yp, z4)
    _, F0 = _edge0(S0, fdummy, *common)
    S1, F1 = _edge_mid(S0, F0, *common)
    S2, F2 = _edge_mid(S1, F1, *common)
    S3, F3, SFp = _edge_sf(S2, F2, *common)
    xp = _final(S3, F3, masses_p, ce16, cf16)

    x = xp[:NV]
    sf = SFp.reshape(NT, SPT_PAD, 3)[:, :SPT].reshape(NS, 3)
    return (x, init_springs, init_rest_lengths, sf)


# speedup vs baseline: 30.5807x; 30.5807x over previous
import functools
import math

import jax
import jax.numpy as jnp
from jax import lax
from jax.experimental import pallas as pl
from jax.experimental.pallas import tpu as pltpu
from jax.experimental.pallas import tpu_sc as plsc

NV = 50000
NS = 1600000
DT = 0.001
DASHPOT = 0.1
DRAG = 0.1
GRAV_Z = -9.8
DAMP = math.exp(-DT * DRAG)

NC, NSC, L = 2, 16, 16
NT = NC * NSC
NV_PAD = 50176
RPT = NV_PAD // NSC
RPT32 = NV_PAD // NT
PR = 224
SPT = NS // NT
CH = 512
NCHUNK = 100
SPT_PAD = CH * NCHUNK
NS_PAD = SPT_PAD * NT
GPC = CH // L

f32 = jnp.float32
i32 = jnp.int32


def _rsqrt(x):
    xb = lax.bitcast_convert_type(x, i32)
    r = lax.bitcast_convert_type(jnp.int32(0x5F3759DF) - (xb >> 1), f32)
    xh = x * 0.5
    for _ in range(3):
        r = r * (1.5 - xh * r * r)
    return r


def _iota():
    return lax.broadcasted_iota(i32, (L,), 0)


def _col(c):
    return jnp.full((L,), c, i32)


def _integrate_group(g, PS, PF0, PF1, PM, CEb, CFb):
    iot = _iota()
    rvec = g * L + iot
    m = PM[pl.ds(g * L, L)]
    ce = CEb[...]
    cf = CFb[...]
    x = [plsc.load_gather(PS, [rvec, _col(c)]) for c in range(3)]
    v = [plsc.load_gather(PS, [rvec, _col(4 + c)]) for c in range(3)]
    f = [plsc.load_gather(PF0, [rvec, _col(c)]) +
         plsc.load_gather(PF1, [rvec, _col(c)]) for c in range(3)]
    f[2] = f[2] + GRAV_Z * m
    dtm = DT / m
    vn = [(v[c] + dtm * f[c]) * DAMP for c in range(3)]
    xn = [x[c] + DT * vn[c] for c in range(3)]
    vz = vn[2]
    mask = (xn[2] < 0.0) & (vz < 0.0)
    tao_sq = vn[0] * vn[0] + vn[1] * vn[1]
    ts_safe = jnp.where(tao_sq > 0.0, tao_sq, 1.0)
    inv_tao = _rsqrt(ts_safe)
    a = jnp.maximum(0.0, 1.0 - cf * (1.0 + ce) * jnp.abs(vz) * inv_tao)
    v3 = [jnp.where(mask, a * vn[0], vn[0]),
          jnp.where(mask, a * vn[1], vn[1]),
          jnp.where(mask, -ce * vz, vz)]
    x3 = [xn[0], xn[1], jnp.where(mask, 0.0, xn[2])]
    return rvec, x3, v3


def _make_edge(mesh, integrate, emit_sf):
    outs = [jax.ShapeDtypeStruct((NV_PAD, 8), f32),
            jax.ShapeDtypeStruct((NC, NV_PAD, 16), f32)]
    if emit_sf:
        outs.append(jax.ShapeDtypeStruct((NS_PAD, 3), f32))

    scratch = (
        pltpu.VMEM_SHARED((NV_PAD, 8), f32),
        pltpu.VMEM_SHARED((NV_PAD, 16), f32),
        pltpu.VMEM((PR, 8), f32),
        pltpu.VMEM((PR, 16), f32),
        pltpu.VMEM((PR, 16), f32),
        pltpu.VMEM((PR,), f32),
        pltpu.VMEM((PR, 8), f32),
        pltpu.VMEM((L,), f32),
        pltpu.VMEM((L,), f32),
        pltpu.VMEM((CH,), i32),
        pltpu.VMEM((CH,), i32),
        pltpu.VMEM((CH,), f32),
        pltpu.VMEM((CH,), f32),
        pltpu.VMEM((CH, 8), f32),
        pltpu.VMEM((CH, 8), f32),
        pltpu.VMEM((CH, 16), f32),
        pltpu.VMEM((CH, 16), f32),
        pltpu.VMEM((CH, 3) if emit_sf else (8, 3), f32),
        pltpu.SemaphoreType.DMA,
        pltpu.SemaphoreType.DMA,
    )

    @functools.partial(pl.kernel, out_type=tuple(outs), mesh=mesh,
                       scratch_types=scratch,
                       compiler_params=pltpu.CompilerParams(
                           needs_layout_passes=False,
                           use_tc_tiling_on_sc=False),
                       name=f"edge_i{int(integrate)}_sf{int(emit_sf)}")
    def call(S_in, F_in, masses, ce16, cf16, idx1v, idx2v, restv, yv, z16,
             *rest):
        if emit_sf:
            S_out, F_out, SF_out = rest[:3]
            scr = rest[3:]
        else:
            S_out, F_out = rest[:2]
            SF_out = None
            scr = rest[2:]
        (S_sh, F_sh, PS, PF0, PF1, PM, PSN, CEb, CFb, I1, I2, RB, YB,
         P1, P2, Q1, Q2, SFB, gsem, ssem) = scr
        cid = lax.axis_index("c")
        sid = lax.axis_index("s")
        wid = cid * NSC + sid
        tb = sid * RPT

        pltpu.sync_copy(z16.at[pl.ds(tb, RPT)], F_sh.at[pl.ds(tb, RPT)])
        pltpu.sync_copy(ce16, CEb)
        pltpu.sync_copy(cf16, CFb)

        if not integrate:
            pltpu.sync_copy(S_in.at[pl.ds(tb, RPT)], S_sh.at[pl.ds(tb, RPT)])
        else:
            for k in range(RPT // PR):
                rows0 = tb + k * PR
                pltpu.sync_copy(S_in.at[pl.ds(rows0, PR)], PS)
                pltpu.sync_copy(F_in.at[0, pl.ds(rows0, PR)], PF0)
                pltpu.sync_copy(F_in.at[1, pl.ds(rows0, PR)], PF1)
                pltpu.sync_copy(masses.at[pl.ds(rows0, PR)], PM)

                @pl.loop(0, PR // L)
                def _grp(g):
                    rvec, x3, v3 = _integrate_group(g, PS, PF0, PF1, PM,
                                                    CEb, CFb)
                    for c in range(3):
                        plsc.store_scatter(PSN, [rvec, _col(c)], x3[c])
                        plsc.store_scatter(PSN, [rvec, _col(4 + c)], v3[c])

                pltpu.sync_copy(PSN, S_sh.at[pl.ds(rows0, PR)])

                @pl.when(cid == 0)
                def _():
                    pltpu.sync_copy(PSN, S_out.at[pl.ds(rows0, PR)])

        pltpu.sync_copy(z16.at[pl.ds(0, CH)], Q1)
        pltpu.sync_copy(z16.at[pl.ds(0, CH)], Q2)

        plsc.subcore_barrier()

        @pl.loop(0, NCHUNK)
        def _chunk(ch):
            base = wid * SPT_PAD + ch * CH
            pltpu.sync_copy(idx1v.at[pl.ds(base, CH)], I1)
            pltpu.sync_copy(idx2v.at[pl.ds(base, CH)], I2)
            pltpu.sync_copy(restv.at[pl.ds(base, CH)], RB)
            pltpu.sync_copy(yv.at[pl.ds(base, CH)], YB)
            g1 = pltpu.async_copy(S_sh.at[I1], P1, gsem)
            g2 = pltpu.async_copy(S_sh.at[I2], P2, gsem)
            g1.wait()
            g2.wait()

            @pl.loop(0, GPC)
            def _grp(g):
                iot = _iota()
                rvec = g * L + iot
                x1 = [plsc.load_gather(P1, [rvec, _col(c)]) for c in range(3)]
                v1 = [plsc.load_gather(P1, [rvec, _col(4 + c)])
                      for c in range(3)]
                x2 = [plsc.load_gather(P2, [rvec, _col(c)]) for c in range(3)]
                v2 = [plsc.load_gather(P2, [rvec, _col(4 + c)])
                      for c in range(3)]
                d0 = x2[0] - x1[0]
                d1 = x2[1] - x1[1]
                d2 = x2[2] - x1[2]
                dn2 = d0 * d0 + d1 * d1 + d2 * d2
                r = _rsqrt(dn2)
                dn = dn2 * r
                rl = RB[pl.ds(g * L, L)]
                e = jnp.exp(YB[pl.ds(g * L, L)])
                s = e * (dn / rl) - e
                vr = ((v2[0] - v1[0]) * d0 + (v2[1] - v1[1]) * d1 +
                      (v2[2] - v1[2]) * d2)
                ts = (s + DASHPOT * vr * r) * r
                fx, fy, fz = ts * d0, ts * d1, ts * d2
                plsc.store_scatter(Q1, [rvec, _col(0)], fx)
                plsc.store_scatter(Q1, [rvec, _col(1)], fy)
                plsc.store_scatter(Q1, [rvec, _col(2)], fz)
                plsc.store_scatter(Q2, [rvec, _col(0)], -fx)
                plsc.store_scatter(Q2, [rvec, _col(1)], -fy)
                plsc.store_scatter(Q2, [rvec, _col(2)], -fz)
                if emit_sf:
                    sr = s * r
                    plsc.store_scatter(SFB, [rvec, _col(0)], sr * d0)
                    plsc.store_scatter(SFB, [rvec, _col(1)], sr * d1)
                    plsc.store_scatter(SFB, [rvec, _col(2)], sr * d2)

            s1 = pltpu.async_copy(Q1, F_sh.at[I1], ssem, add=True)
            s2 = pltpu.async_copy(Q2, F_sh.at[I2], ssem, add=True)
            if emit_sf:
                pltpu.sync_copy(SFB, SF_out.at[pl.ds(base, CH)])
            s1.wait()
            s2.wait()

        plsc.subcore_barrier()
        pltpu.sync_copy(F_sh.at[pl.ds(tb, RPT)],
                        F_out.at[cid, pl.ds(tb, RPT)])

    return call


def _make_final(mesh):
    scratch = (
        pltpu.VMEM((PR, 8), f32),
        pltpu.VMEM((PR, 16), f32),
        pltpu.VMEM((PR, 16), f32),
        pltpu.VMEM((PR,), f32),
        pltpu.VMEM((PR, 3), f32),
        pltpu.VMEM((L,), f32),
        pltpu.VMEM((L,), f32),
    )

    @functools.partial(pl.kernel,
                       out_type=jax.ShapeDtypeStruct((NV_PAD, 3), f32),
                       mesh=mesh, scratch_types=scratch,
                       compiler_params=pltpu.CompilerParams(
                           needs_layout_passes=False,
                           use_tc_tiling_on_sc=False),
                       name="final_integrate")
    def call(S_in, F_in, masses, ce16, cf16, x_out,
             PS, PF0, PF1, PM, XB, CEb, CFb):
        cid = lax.axis_index("c")
        sid = lax.axis_index("s")
        wid = cid * NSC + sid
        tb = wid * RPT32
        pltpu.sync_copy(ce16, CEb)
        pltpu.sync_copy(cf16, CFb)
        for k in range(RPT32 // PR):
            rows0 = tb + k * PR
            pltpu.sync_copy(S_in.at[pl.ds(rows0, PR)], PS)
            pltpu.sync_copy(F_in.at[0, pl.ds(rows0, PR)], PF0)
            pltpu.sync_copy(F_in.at[1, pl.ds(rows0, PR)], PF1)
            pltpu.sync_copy(masses.at[pl.ds(rows0, PR)], PM)

            @pl.loop(0, PR // L)
            def _grp(g):
                rvec, x3, _ = _integrate_group(g, PS, PF0, PF1, PM, CEb, CFb)
                for c in range(3):
                    plsc.store_scatter(XB, [rvec, _col(c)], x3[c])

            pltpu.sync_copy(XB, x_out.at[pl.ds(rows0, PR)])

    return call


_CALLS = {}


def _get_calls():
    if not _CALLS:
        mesh = plsc.VectorSubcoreMesh(core_axis_name="c", subcore_axis_name="s",
                                      num_cores=NC, num_subcores=NSC)
        _CALLS["edge0"] = _make_edge(mesh, integrate=False, emit_sf=False)
        _CALLS["edge_mid"] = _make_edge(mesh, integrate=True, emit_sf=False)
        _CALLS["edge_sf"] = _make_edge(mesh, integrate=True, emit_sf=True)
        _CALLS["final"] = _make_final(mesh)
    return _CALLS


def kernel(init_vertices, init_springs, init_rest_lengths, init_masses,
           spring_Y, collide_elas, collide_fric):
    S0 = jnp.zeros((NV_PAD, 8), f32).at[:NV, 0:3].set(init_vertices)
    masses_p = jnp.ones((NV_PAD,), f32).at[:NV].set(init_masses)
    idx1 = init_springs[:, 0]
    idx2 = init_springs[:, 1]

    def pad_springs(a, fill):
        a2 = a.reshape(NT, SPT)
        padblock = jnp.full((NT, SPT_PAD - SPT), fill, a.dtype)
        return jnp.concatenate([a2, padblock], axis=1).reshape(-1)

    i1p = pad_springs(idx1, NV)
    i2p = pad_springs(idx2, NV)
    restp = pad_springs(init_rest_lengths, 1.0)
    yp = pad_springs(spring_Y, 0.0)
    z16 = jnp.zeros((NV_PAD, 16), f32)
    ce16 = jnp.full((L,), collide_elas, f32)
    cf16 = jnp.full((L,), collide_fric, f32)
    fdummy = jnp.zeros((NC, NV_PAD, 16), f32)

    calls = _get_calls()
    common = (masses_p, ce16, cf16, i1p, i2p, restp, yp, z16)
    _, F0 = calls["edge0"](S0, fdummy, *common)
    S1, F1 = calls["edge_mid"](S0, F0, *common)
    S2, F2 = calls["edge_mid"](S1, F1, *common)
    S3, F3, SFp = calls["edge_sf"](S2, F2, *common)
    xp = calls["final"](S3, F3, masses_p, ce16, cf16)

    x = xp[:NV]
    sf = SFp.reshape(NT, SPT_PAD, 3)[:, :SPT].reshape(NS, 3)
    return (x, init_springs, init_rest_lengths, sf)


# speedup vs baseline: 36.2289x; 1.1847x over previous
import functools
import math

import jax
import jax.numpy as jnp
from jax import lax
from jax.experimental import pallas as pl
from jax.experimental.pallas import tpu as pltpu
from jax.experimental.pallas import tpu_sc as plsc

NV = 50000
NS = 1600000
DT = 0.001
DASHPOT = 0.1
DRAG = 0.1
GRAV_Z = -9.8
DAMP = math.exp(-DT * DRAG)

NC, NSC, L = 2, 16, 16
NT = NC * NSC
NV_PAD = 50176
RPT = NV_PAD // NSC
RPT32 = NV_PAD // NT
PR = 224
SPT = NS // NT
CH = 512
NCHUNK = 100
SPT_PAD = CH * NCHUNK
NS_PAD = SPT_PAD * NT
GPC = CH // L

f32 = jnp.float32
i32 = jnp.int32


def _rsqrt(x):
    xb = lax.bitcast_convert_type(x, i32)
    r = lax.bitcast_convert_type(jnp.int32(0x5F3759DF) - (xb >> 1), f32)
    xh = x * 0.5
    for _ in range(3):
        r = r * (1.5 - xh * r * r)
    return r


def _iota():
    return lax.broadcasted_iota(i32, (L,), 0)


def _col(c):
    return jnp.full((L,), c, i32)


def _integrate_group(g, PS, PF0, PF1, PM, CEb, CFb):
    iot = _iota()
    rvec = g * L + iot
    m = PM[pl.ds(g * L, L)]
    ce = CEb[...]
    cf = CFb[...]
    x = [plsc.load_gather(PS, [rvec, _col(c)]) for c in range(3)]
    v = [plsc.load_gather(PS, [rvec, _col(4 + c)]) for c in range(3)]
    f = [plsc.load_gather(PF0, [rvec, _col(c)]) +
         plsc.load_gather(PF1, [rvec, _col(c)]) for c in range(3)]
    f[2] = f[2] + GRAV_Z * m
    dtm = DT / m
    vn = [(v[c] + dtm * f[c]) * DAMP for c in range(3)]
    xn = [x[c] + DT * vn[c] for c in range(3)]
    vz = vn[2]
    mask = (xn[2] < 0.0) & (vz < 0.0)
    tao_sq = vn[0] * vn[0] + vn[1] * vn[1]
    ts_safe = jnp.where(tao_sq > 0.0, tao_sq, 1.0)
    inv_tao = _rsqrt(ts_safe)
    a = jnp.maximum(0.0, 1.0 - cf * (1.0 + ce) * jnp.abs(vz) * inv_tao)
    v3 = [jnp.where(mask, a * vn[0], vn[0]),
          jnp.where(mask, a * vn[1], vn[1]),
          jnp.where(mask, -ce * vz, vz)]
    x3 = [xn[0], xn[1], jnp.where(mask, 0.0, xn[2])]
    return rvec, x3, v3


def _make_edge(mesh, integrate, emit_sf):
    outs = [jax.ShapeDtypeStruct((NV_PAD, 8), f32),
            jax.ShapeDtypeStruct((NC, NV_PAD, 16), f32)]
    if emit_sf:
        outs.append(jax.ShapeDtypeStruct((NS_PAD, 3), f32))

    scratch = (
        pltpu.VMEM_SHARED((NV_PAD, 8), f32),
        pltpu.VMEM_SHARED((NV_PAD, 16), f32),
        pltpu.VMEM((PR, 8), f32),
        pltpu.VMEM((PR, 16), f32),
        pltpu.VMEM((PR, 16), f32),
        pltpu.VMEM((PR,), f32),
        pltpu.VMEM((PR, 8), f32),
        pltpu.VMEM((L,), f32),
        pltpu.VMEM((L,), f32),
        pltpu.VMEM((CH,), i32),
        pltpu.VMEM((CH,), i32),
        pltpu.VMEM((CH,), f32),
        pltpu.VMEM((CH,), f32),
        pltpu.VMEM((CH, 8), f32),
        pltpu.VMEM((CH, 8), f32),
        pltpu.VMEM((CH, 16), f32),
        pltpu.VMEM((CH, 16), f32),
        pltpu.VMEM((CH, 3) if emit_sf else (8, 3), f32),
        pltpu.SemaphoreType.DMA,
        pltpu.SemaphoreType.DMA,
        pltpu.SemaphoreType.DMA,
    )

    @functools.partial(pl.kernel, out_type=tuple(outs), mesh=mesh,
                       scratch_types=scratch,
                       compiler_params=pltpu.CompilerParams(
                           needs_layout_passes=False,
                           use_tc_tiling_on_sc=False),
                       name=f"edge_i{int(integrate)}_sf{int(emit_sf)}")
    def call(S_in, F_in, masses, ce16, cf16, idx1v, idx2v, kv, ev, z16,
             *rest):
        if emit_sf:
            S_out, F_out, SF_out = rest[:3]
            scr = rest[3:]
        else:
            S_out, F_out = rest[:2]
            SF_out = None
            scr = rest[2:]
        (S_sh, F_sh, PS, PF0, PF1, PM, PSN, CEb, CFb, I1, I2, KB, EB,
         P1, P2, Q1, Q2, SFB, gsem, ssem, psem) = scr
        cid = lax.axis_index("c")
        sid = lax.axis_index("s")
        wid = cid * NSC + sid
        tb = sid * RPT

        pltpu.sync_copy(z16.at[pl.ds(tb, RPT)], F_sh.at[pl.ds(tb, RPT)])
        pltpu.sync_copy(ce16, CEb)
        pltpu.sync_copy(cf16, CFb)

        if not integrate:
            pltpu.sync_copy(S_in.at[pl.ds(tb, RPT)], S_sh.at[pl.ds(tb, RPT)])
        else:
            for k in range(RPT // PR):
                rows0 = tb + k * PR
                c1 = pltpu.async_copy(S_in.at[pl.ds(rows0, PR)], PS, psem)
                c2 = pltpu.async_copy(F_in.at[0, pl.ds(rows0, PR)], PF0, psem)
                c3 = pltpu.async_copy(F_in.at[1, pl.ds(rows0, PR)], PF1, psem)
                c4 = pltpu.async_copy(masses.at[pl.ds(rows0, PR)], PM, psem)
                c1.wait()
                c2.wait()
                c3.wait()
                c4.wait()

                @pl.loop(0, PR // L)
                def _grp(g):
                    rvec, x3, v3 = _integrate_group(g, PS, PF0, PF1, PM,
                                                    CEb, CFb)
                    for c in range(3):
                        plsc.store_scatter(PSN, [rvec, _col(c)], x3[c])
                        plsc.store_scatter(PSN, [rvec, _col(4 + c)], v3[c])

                pltpu.sync_copy(PSN, S_sh.at[pl.ds(rows0, PR)])

                @pl.when(cid == 0)
                def _():
                    pltpu.sync_copy(PSN, S_out.at[pl.ds(rows0, PR)])

        pltpu.sync_copy(z16.at[pl.ds(0, CH)], Q1)
        pltpu.sync_copy(z16.at[pl.ds(0, CH)], Q2)

        plsc.subcore_barrier()

        @pl.loop(0, NCHUNK)
        def _chunk(ch):
            base = wid * SPT_PAD + ch * CH
            c1 = pltpu.async_copy(idx1v.at[pl.ds(base, CH)], I1, psem)
            c2 = pltpu.async_copy(idx2v.at[pl.ds(base, CH)], I2, psem)
            c3 = pltpu.async_copy(kv.at[pl.ds(base, CH)], KB, psem)
            c4 = pltpu.async_copy(ev.at[pl.ds(base, CH)], EB, psem)
            c1.wait()
            c2.wait()
            c3.wait()
            c4.wait()
            g1 = pltpu.async_copy(S_sh.at[I1], P1, gsem)
            g2 = pltpu.async_copy(S_sh.at[I2], P2, gsem)
            g1.wait()
            g2.wait()

            @pl.loop(0, GPC)
            def _grp(g):
                iot = _iota()
                rvec = g * L + iot
                x1 = [plsc.load_gather(P1, [rvec, _col(c)]) for c in range(3)]
                v1 = [plsc.load_gather(P1, [rvec, _col(4 + c)])
                      for c in range(3)]
                x2 = [plsc.load_gather(P2, [rvec, _col(c)]) for c in range(3)]
                v2 = [plsc.load_gather(P2, [rvec, _col(4 + c)])
                      for c in range(3)]
                d0 = x2[0] - x1[0]
                d1 = x2[1] - x1[1]
                d2 = x2[2] - x1[2]
                dn2 = d0 * d0 + d1 * d1 + d2 * d2
                r = _rsqrt(dn2)
                krl = KB[pl.ds(g * L, L)]
                e = EB[pl.ds(g * L, L)]
                sr = krl - e * r
                vr = ((v2[0] - v1[0]) * d0 + (v2[1] - v1[1]) * d1 +
                      (v2[2] - v1[2]) * d2)
                ts = sr + DASHPOT * (vr * r) * r
                fx, fy, fz = ts * d0, ts * d1, ts * d2
                plsc.store_scatter(Q1, [rvec, _col(0)], fx)
                plsc.store_scatter(Q1, [rvec, _col(1)], fy)
                plsc.store_scatter(Q1, [rvec, _col(2)], fz)
                plsc.store_scatter(Q2, [rvec, _col(0)], -fx)
                plsc.store_scatter(Q2, [rvec, _col(1)], -fy)
                plsc.store_scatter(Q2, [rvec, _col(2)], -fz)
                if emit_sf:
                    plsc.store_scatter(SFB, [rvec, _col(0)], sr * d0)
                    plsc.store_scatter(SFB, [rvec, _col(1)], sr * d1)
                    plsc.store_scatter(SFB, [rvec, _col(2)], sr * d2)

            s1 = pltpu.async_copy(Q1, F_sh.at[I1], ssem, add=True)
            s2 = pltpu.async_copy(Q2, F_sh.at[I2], ssem, add=True)
            if emit_sf:
                pltpu.sync_copy(SFB, SF_out.at[pl.ds(base, CH)])
            s1.wait()
            s2.wait()

        plsc.subcore_barrier()
        pltpu.sync_copy(F_sh.at[pl.ds(tb, RPT)],
                        F_out.at[cid, pl.ds(tb, RPT)])

    return call


def _make_final(mesh):
    scratch = (
        pltpu.VMEM((PR, 8), f32),
        pltpu.VMEM((PR, 16), f32),
        pltpu.VMEM((PR, 16), f32),
        pltpu.VMEM((PR,), f32),
        pltpu.VMEM((PR, 3), f32),
        pltpu.VMEM((L,), f32),
        pltpu.VMEM((L,), f32),
    )

    @functools.partial(pl.kernel,
                       out_type=jax.ShapeDtypeStruct((NV_PAD, 3), f32),
                       mesh=mesh, scratch_types=scratch,
                       compiler_params=pltpu.CompilerParams(
                           needs_layout_passes=False,
                           use_tc_tiling_on_sc=False),
                       name="final_integrate")
    def call(S_in, F_in, masses, ce16, cf16, x_out,
             PS, PF0, PF1, PM, XB, CEb, CFb):
        cid = lax.axis_index("c")
        sid = lax.axis_index("s")
        wid = cid * NSC + sid
        tb = wid * RPT32
        pltpu.sync_copy(ce16, CEb)
        pltpu.sync_copy(cf16, CFb)
        for k in range(RPT32 // PR):
            rows0 = tb + k * PR
            pltpu.sync_copy(S_in.at[pl.ds(rows0, PR)], PS)
            pltpu.sync_copy(F_in.at[0, pl.ds(rows0, PR)], PF0)
            pltpu.sync_copy(F_in.at[1, pl.ds(rows0, PR)], PF1)
            pltpu.sync_copy(masses.at[pl.ds(rows0, PR)], PM)

            @pl.loop(0, PR // L)
            def _grp(g):
                rvec, x3, _ = _integrate_group(g, PS, PF0, PF1, PM, CEb, CFb)
                for c in range(3):
                    plsc.store_scatter(XB, [rvec, _col(c)], x3[c])

            pltpu.sync_copy(XB, x_out.at[pl.ds(rows0, PR)])

    return call


_CALLS = {}


def _get_calls():
    if not _CALLS:
        mesh = plsc.VectorSubcoreMesh(core_axis_name="c", subcore_axis_name="s",
                                      num_cores=NC, num_subcores=NSC)
        _CALLS["edge0"] = _make_edge(mesh, integrate=False, emit_sf=False)
        _CALLS["edge_mid"] = _make_edge(mesh, integrate=True, emit_sf=False)
        _CALLS["edge_sf"] = _make_edge(mesh, integrate=True, emit_sf=True)
        _CALLS["final"] = _make_final(mesh)
    return _CALLS


def kernel(init_vertices, init_springs, init_rest_lengths, init_masses,
           spring_Y, collide_elas, collide_fric):
    S0 = jnp.zeros((NV_PAD, 8), f32).at[:NV, 0:3].set(init_vertices)
    masses_p = jnp.ones((NV_PAD,), f32).at[:NV].set(init_masses)
    idx1 = init_springs[:, 0]
    idx2 = init_springs[:, 1]

    def pad_springs(a, fill):
        a2 = a.reshape(NT, SPT)
        padblock = jnp.full((NT, SPT_PAD - SPT), fill, a.dtype)
        return jnp.concatenate([a2, padblock], axis=1).reshape(-1)

    i1p = pad_springs(idx1, NV)
    i2p = pad_springs(idx2, NV)
    e_spring = jnp.exp(spring_Y)
    kp = pad_springs(e_spring / init_rest_lengths, 0.0)
    ep = pad_springs(e_spring, 0.0)
    z16 = jnp.zeros((NV_PAD, 16), f32)
    ce16 = jnp.full((L,), collide_elas, f32)
    cf16 = jnp.full((L,), collide_fric, f32)
    fdummy = jnp.zeros((NC, NV_PAD, 16), f32)

    calls = _get_calls()
    common = (masses_p, ce16, cf16, i1p, i2p, kp, ep, z16)
    _, F0 = calls["edge0"](S0, fdummy, *common)
    S1, F1 = calls["edge_mid"](S0, F0, *common)
    S2, F2 = calls["edge_mid"](S1, F1, *common)
    S3, F3, SFp = calls["edge_sf"](S2, F2, *common)
    xp = calls["final"](S3, F3, masses_p, ce16, cf16)

    x = xp[:NV]
    sf = SFp.reshape(NT, SPT_PAD, 3)[:, :SPT].reshape(NS, 3)
    return (x, init_springs, init_rest_lengths, sf)


# speedup vs baseline: 36.9846x; 1.0209x over previous
import functools
import math

import jax
import jax.numpy as jnp
from jax import lax
from jax.experimental import pallas as pl
from jax.experimental.pallas import tpu as pltpu
from jax.experimental.pallas import tpu_sc as plsc

NV = 50000
NS = 1600000
DT = 0.001
DASHPOT = 0.1
DRAG = 0.1
GRAV_Z = -9.8
DAMP = math.exp(-DT * DRAG)

NC, NSC, L = 2, 16, 16
NT = NC * NSC
NV_PAD = 50176
RPT = NV_PAD // NSC
RPT32 = NV_PAD // NT
PR = 224
SPT = NS // NT
CH = 512
NCHUNK = 100
SPT_PAD = CH * NCHUNK
NS_PAD = SPT_PAD * NT
GPC = CH // L

f32 = jnp.float32
i32 = jnp.int32


def _rsqrt(x):
    xb = lax.bitcast_convert_type(x, i32)
    r = lax.bitcast_convert_type(jnp.int32(0x5F3759DF) - (xb >> 1), f32)
    xh = x * 0.5
    for _ in range(2):
        r = r * (1.5 - xh * r * r)
    return r


def _iota():
    return lax.broadcasted_iota(i32, (L,), 0)


def _col(c):
    return jnp.full((L,), c, i32)


def _integrate_group(g, PS, PF0, PF1, PM, CEb, CFb):
    iot = _iota()
    rvec = g * L + iot
    m = PM[pl.ds(g * L, L)]
    ce = CEb[...]
    cf = CFb[...]
    x = [plsc.load_gather(PS, [rvec, _col(c)]) for c in range(3)]
    v = [plsc.load_gather(PS, [rvec, _col(4 + c)]) for c in range(3)]
    f = [plsc.load_gather(PF0, [rvec, _col(c)]) +
         plsc.load_gather(PF1, [rvec, _col(c)]) for c in range(3)]
    f[2] = f[2] + GRAV_Z * m
    dtm = DT / m
    vn = [(v[c] + dtm * f[c]) * DAMP for c in range(3)]
    xn = [x[c] + DT * vn[c] for c in range(3)]
    vz = vn[2]
    mask = (xn[2] < 0.0) & (vz < 0.0)
    tao_sq = vn[0] * vn[0] + vn[1] * vn[1]
    ts_safe = jnp.where(tao_sq > 0.0, tao_sq, 1.0)
    inv_tao = _rsqrt(ts_safe)
    a = jnp.maximum(0.0, 1.0 - cf * (1.0 + ce) * jnp.abs(vz) * inv_tao)
    v3 = [jnp.where(mask, a * vn[0], vn[0]),
          jnp.where(mask, a * vn[1], vn[1]),
          jnp.where(mask, -ce * vz, vz)]
    x3 = [xn[0], xn[1], jnp.where(mask, 0.0, xn[2])]
    return rvec, x3, v3


def _make_edge(mesh, integrate, emit_sf):
    outs = [jax.ShapeDtypeStruct((NV_PAD, 8), f32),
            jax.ShapeDtypeStruct((NC, NV_PAD, 16), f32)]
    if emit_sf:
        outs.append(jax.ShapeDtypeStruct((NS_PAD, 3), f32))

    scratch = (
        pltpu.VMEM_SHARED((NV_PAD, 8), f32),
        pltpu.VMEM_SHARED((NV_PAD, 16), f32),
        pltpu.VMEM((PR, 8), f32),
        pltpu.VMEM((PR, 16), f32),
        pltpu.VMEM((PR, 16), f32),
        pltpu.VMEM((PR,), f32),
        pltpu.VMEM((PR, 8), f32),
        pltpu.VMEM((L,), f32),
        pltpu.VMEM((L,), f32),
        pltpu.VMEM((CH,), i32),
        pltpu.VMEM((CH,), i32),
        pltpu.VMEM((CH,), f32),
        pltpu.VMEM((CH,), f32),
        pltpu.VMEM((CH, 8), f32),
        pltpu.VMEM((CH, 8), f32),
        pltpu.VMEM((CH, 16), f32),
        pltpu.VMEM((CH, 16), f32),
        pltpu.VMEM((CH, 3) if emit_sf else (8, 3), f32),
        pltpu.SemaphoreType.DMA,
        pltpu.SemaphoreType.DMA,
        pltpu.SemaphoreType.DMA,
    )

    @functools.partial(pl.kernel, out_type=tuple(outs), mesh=mesh,
                       scratch_types=scratch,
                       compiler_params=pltpu.CompilerParams(
                           needs_layout_passes=False,
                           use_tc_tiling_on_sc=False),
                       name=f"edge_i{int(integrate)}_sf{int(emit_sf)}")
    def call(S_in, F_in, masses, ce16, cf16, idx1v, idx2v, kv, ev, z16,
             *rest):
        if emit_sf:
            S_out, F_out, SF_out = rest[:3]
            scr = rest[3:]
        else:
            S_out, F_out = rest[:2]
            SF_out = None
            scr = rest[2:]
        (S_sh, F_sh, PS, PF0, PF1, PM, PSN, CEb, CFb, I1, I2, KB, EB,
         P1, P2, Q1, Q2, SFB, gsem, ssem, psem) = scr
        cid = lax.axis_index("c")
        sid = lax.axis_index("s")
        wid = cid * NSC + sid
        tb = sid * RPT

        pltpu.sync_copy(z16.at[pl.ds(tb, RPT)], F_sh.at[pl.ds(tb, RPT)])
        pltpu.sync_copy(ce16, CEb)
        pltpu.sync_copy(cf16, CFb)

        if not integrate:
            pltpu.sync_copy(S_in.at[pl.ds(tb, RPT)], S_sh.at[pl.ds(tb, RPT)])
        else:
            for k in range(RPT // PR):
                rows0 = tb + k * PR
                c1 = pltpu.async_copy(S_in.at[pl.ds(rows0, PR)], PS, psem)
                c2 = pltpu.async_copy(F_in.at[0, pl.ds(rows0, PR)], PF0, psem)
                c3 = pltpu.async_copy(F_in.at[1, pl.ds(rows0, PR)], PF1, psem)
                c4 = pltpu.async_copy(masses.at[pl.ds(rows0, PR)], PM, psem)
                c1.wait()
                c2.wait()
                c3.wait()
                c4.wait()

                @pl.loop(0, PR // L)
                def _grp(g):
                    rvec, x3, v3 = _integrate_group(g, PS, PF0, PF1, PM,
                                                    CEb, CFb)
                    for c in range(3):
                        plsc.store_scatter(PSN, [rvec, _col(c)], x3[c])
                        plsc.store_scatter(PSN, [rvec, _col(4 + c)], v3[c])

                pltpu.sync_copy(PSN, S_sh.at[pl.ds(rows0, PR)])

                @pl.when(cid == 0)
                def _():
                    pltpu.sync_copy(PSN, S_out.at[pl.ds(rows0, PR)])

        pltpu.sync_copy(z16.at[pl.ds(0, CH)], Q1)
        pltpu.sync_copy(z16.at[pl.ds(0, CH)], Q2)

        plsc.subcore_barrier()

        @pl.loop(0, NCHUNK)
        def _chunk(ch):
            base = wid * SPT_PAD + ch * CH
            c1 = pltpu.async_copy(idx1v.at[pl.ds(base, CH)], I1, psem)
            c2 = pltpu.async_copy(idx2v.at[pl.ds(base, CH)], I2, psem)
            c3 = pltpu.async_copy(kv.at[pl.ds(base, CH)], KB, psem)
            c4 = pltpu.async_copy(ev.at[pl.ds(base, CH)], EB, psem)
            c1.wait()
            c2.wait()
            c3.wait()
            c4.wait()
            g1 = pltpu.async_copy(S_sh.at[I1], P1, gsem)
            g2 = pltpu.async_copy(S_sh.at[I2], P2, gsem)
            g1.wait()
            g2.wait()

            @pl.loop(0, GPC // 2)
            def _grp(gh):
                for gb in range(2):
                    g = gh * 2 + gb
                    iot = _iota()
                    rvec = g * L + iot
                    x1 = [plsc.load_gather(P1, [rvec, _col(c)])
                          for c in range(3)]
                    v1 = [plsc.load_gather(P1, [rvec, _col(4 + c)])
                          for c in range(3)]
                    x2 = [plsc.load_gather(P2, [rvec, _col(c)])
                          for c in range(3)]
                    v2 = [plsc.load_gather(P2, [rvec, _col(4 + c)])
                          for c in range(3)]
                    d0 = x2[0] - x1[0]
                    d1 = x2[1] - x1[1]
                    d2 = x2[2] - x1[2]
                    dn2 = d0 * d0 + d1 * d1 + d2 * d2
                    r = _rsqrt(dn2)
                    krl = KB[pl.ds(g * L, L)]
                    e = EB[pl.ds(g * L, L)]
                    sr = krl - e * r
                    vr = ((v2[0] - v1[0]) * d0 + (v2[1] - v1[1]) * d1 +
                          (v2[2] - v1[2]) * d2)
                    ts = sr + DASHPOT * (vr * r) * r
                    fx, fy, fz = ts * d0, ts * d1, ts * d2
                    plsc.store_scatter(Q1, [rvec, _col(0)], fx)
                    plsc.store_scatter(Q1, [rvec, _col(1)], fy)
                    plsc.store_scatter(Q1, [rvec, _col(2)], fz)
                    plsc.store_scatter(Q2, [rvec, _col(0)], -fx)
                    plsc.store_scatter(Q2, [rvec, _col(1)], -fy)
                    plsc.store_scatter(Q2, [rvec, _col(2)], -fz)
                    if emit_sf:
                        plsc.store_scatter(SFB, [rvec, _col(0)], sr * d0)
                        plsc.store_scatter(SFB, [rvec, _col(1)], sr * d1)
                        plsc.store_scatter(SFB, [rvec, _col(2)], sr * d2)

            s1 = pltpu.async_copy(Q1, F_sh.at[I1], ssem, add=True)
            s2 = pltpu.async_copy(Q2, F_sh.at[I2], ssem, add=True)
            if emit_sf:
                pltpu.sync_copy(SFB, SF_out.at[pl.ds(base, CH)])
            s1.wait()
            s2.wait()

        plsc.subcore_barrier()
        pltpu.sync_copy(F_sh.at[pl.ds(tb, RPT)],
                        F_out.at[cid, pl.ds(tb, RPT)])

    return call


def _make_final(mesh):
    scratch = (
        pltpu.VMEM((PR, 8), f32),
        pltpu.VMEM((PR, 16), f32),
        pltpu.VMEM((PR, 16), f32),
        pltpu.VMEM((PR,), f32),
        pltpu.VMEM((PR, 3), f32),
        pltpu.VMEM((L,), f32),
        pltpu.VMEM((L,), f32),
    )

    @functools.partial(pl.kernel,
                       out_type=jax.ShapeDtypeStruct((NV_PAD, 3), f32),
                       mesh=mesh, scratch_types=scratch,
                       compiler_params=pltpu.CompilerParams(
                           needs_layout_passes=False,
                           use_tc_tiling_on_sc=False),
                       name="final_integrate")
    def call(S_in, F_in, masses, ce16, cf16, x_out,
             PS, PF0, PF1, PM, XB, CEb, CFb):
        cid = lax.axis_index("c")
        sid = lax.axis_index("s")
        wid = cid * NSC + sid
        tb = wid * RPT32
        pltpu.sync_copy(ce16, CEb)
        pltpu.sync_copy(cf16, CFb)
        for k in range(RPT32 // PR):
            rows0 = tb + k * PR
            pltpu.sync_copy(S_in.at[pl.ds(rows0, PR)], PS)
            pltpu.sync_copy(F_in.at[0, pl.ds(rows0, PR)], PF0)
            pltpu.sync_copy(F_in.at[1, pl.ds(rows0, PR)], PF1)
            pltpu.sync_copy(masses.at[pl.ds(rows0, PR)], PM)

            @pl.loop(0, PR // L)
            def _grp(g):
                rvec, x3, _ = _integrate_group(g, PS, PF0, PF1, PM, CEb, CFb)
                for c in range(3):
                    plsc.store_scatter(XB, [rvec, _col(c)], x3[c])

            pltpu.sync_copy(XB, x_out.at[pl.ds(rows0, PR)])

    return call


_CALLS = {}


def _get_calls():
    if not _CALLS:
        mesh = plsc.VectorSubcoreMesh(core_axis_name="c", subcore_axis_name="s",
                                      num_cores=NC, num_subcores=NSC)
        _CALLS["edge0"] = _make_edge(mesh, integrate=False, emit_sf=False)
        _CALLS["edge_mid"] = _make_edge(mesh, integrate=True, emit_sf=False)
        _CALLS["edge_sf"] = _make_edge(mesh, integrate=True, emit_sf=True)
        _CALLS["final"] = _make_final(mesh)
    return _CALLS


def kernel(init_vertices, init_springs, init_rest_lengths, init_masses,
           spring_Y, collide_elas, collide_fric):
    S0 = jnp.zeros((NV_PAD, 8), f32).at[:NV, 0:3].set(init_vertices)
    masses_p = jnp.ones((NV_PAD,), f32).at[:NV].set(init_masses)
    idx1 = init_springs[:, 0]
    idx2 = init_springs[:, 1]

    def pad_springs(a, fill):
        a2 = a.reshape(NT, SPT)
        padblock = jnp.full((NT, SPT_PAD - SPT), fill, a.dtype)
        return jnp.concatenate([a2, padblock], axis=1).reshape(-1)

    i1p = pad_springs(idx1, NV)
    i2p = pad_springs(idx2, NV)
    e_spring = jnp.exp(spring_Y)
    kp = pad_springs(e_spring / init_rest_lengths, 0.0)
    ep = pad_springs(e_spring, 0.0)
    z16 = jnp.zeros((NV_PAD, 16), f32)
    ce16 = jnp.full((L,), collide_elas, f32)
    cf16 = jnp.full((L,), collide_fric, f32)
    fdummy = jnp.zeros((NC, NV_PAD, 16), f32)

    calls = _get_calls()
    common = (masses_p, ce16, cf16, i1p, i2p, kp, ep, z16)
    _, F0 = calls["edge0"](S0, fdummy, *common)
    S1, F1 = calls["edge_mid"](S0, F0, *common)
    S2, F2 = calls["edge_mid"](S1, F1, *common)
    S3, F3, SFp = calls["edge_sf"](S2, F2, *common)
    xp = calls["final"](S3, F3, masses_p, ce16, cf16)

    x = xp[:NV]
    sf = SFp.reshape(NT, SPT_PAD, 3)[:, :SPT].reshape(NS, 3)
    return (x, init_springs, init_rest_lengths, sf)


# speedup vs baseline: 43.2088x; 1.1683x over previous
import functools
import math

import jax
import jax.numpy as jnp
from jax import lax
from jax.experimental import pallas as pl
from jax.experimental.pallas import tpu as pltpu
from jax.experimental.pallas import tpu_sc as plsc

NV = 50000
NS = 1600000
DT = 0.001
DASHPOT = 0.1
DRAG = 0.1
GRAV_Z = -9.8
DAMP = math.exp(-DT * DRAG)

NC, NSC, L = 2, 16, 16
NT = NC * NSC
NV_PAD = 50176
RPT = NV_PAD // NSC
RPT32 = NV_PAD // NT
PR = 224
SPT = NS // NT
CH = 256
NCHUNK = 200
SPT_PAD = CH * NCHUNK
NS_PAD = SPT_PAD * NT
GPC = CH // L

f32 = jnp.float32
i32 = jnp.int32


def _rsqrt(x):
    xb = lax.bitcast_convert_type(x, i32)
    r = lax.bitcast_convert_type(jnp.int32(0x5F3759DF) - (xb >> 1), f32)
    xh = x * 0.5
    for _ in range(2):
        r = r * (1.5 - xh * r * r)
    return r


def _iota():
    return lax.broadcasted_iota(i32, (L,), 0)


def _col(c):
    return jnp.full((L,), c, i32)


def _integrate_group(g, PS, PF0, PF1, PM, CEb, CFb):
    iot = _iota()
    rvec = g * L + iot
    m = PM[pl.ds(g * L, L)]
    ce = CEb[...]
    cf = CFb[...]
    x = [plsc.load_gather(PS, [rvec, _col(c)]) for c in range(3)]
    v = [plsc.load_gather(PS, [rvec, _col(4 + c)]) for c in range(3)]
    f = [plsc.load_gather(PF0, [rvec, _col(c)]) +
         plsc.load_gather(PF1, [rvec, _col(c)]) for c in range(3)]
    f[2] = f[2] + GRAV_Z * m
    dtm = DT / m
    vn = [(v[c] + dtm * f[c]) * DAMP for c in range(3)]
    xn = [x[c] + DT * vn[c] for c in range(3)]
    vz = vn[2]
    mask = (xn[2] < 0.0) & (vz < 0.0)
    tao_sq = vn[0] * vn[0] + vn[1] * vn[1]
    ts_safe = jnp.where(tao_sq > 0.0, tao_sq, 1.0)
    inv_tao = _rsqrt(ts_safe)
    a = jnp.maximum(0.0, 1.0 - cf * (1.0 + ce) * jnp.abs(vz) * inv_tao)
    v3 = [jnp.where(mask, a * vn[0], vn[0]),
          jnp.where(mask, a * vn[1], vn[1]),
          jnp.where(mask, -ce * vz, vz)]
    x3 = [xn[0], xn[1], jnp.where(mask, 0.0, xn[2])]
    return rvec, x3, v3


def _make_edge(mesh, integrate, emit_sf):
    outs = [jax.ShapeDtypeStruct((NV_PAD, 8), f32),
            jax.ShapeDtypeStruct((NC, NV_PAD, 16), f32)]
    if emit_sf:
        outs.append(jax.ShapeDtypeStruct((NS_PAD, 3), f32))

    scratch = (
        pltpu.VMEM_SHARED((NV_PAD, 8), f32),
        pltpu.VMEM_SHARED((NV_PAD, 16), f32),
        pltpu.VMEM((PR, 8), f32),
        pltpu.VMEM((PR, 16), f32),
        pltpu.VMEM((PR, 16), f32),
        pltpu.VMEM((PR,), f32),
        pltpu.VMEM((PR, 8), f32),
        pltpu.VMEM((L,), f32),
        pltpu.VMEM((L,), f32),
        *[pltpu.VMEM((CH,), i32) for _ in range(4)],
        *[pltpu.VMEM((CH,), i32) for _ in range(4)],
        *[pltpu.VMEM((CH,), f32) for _ in range(4)],
        *[pltpu.VMEM((CH,), f32) for _ in range(4)],
        *[pltpu.VMEM((CH, 8), f32) for _ in range(2)],
        *[pltpu.VMEM((CH, 8), f32) for _ in range(2)],
        *[pltpu.VMEM((CH, 16), f32) for _ in range(2)],
        *[pltpu.VMEM((CH, 16), f32) for _ in range(2)],
        *[pltpu.VMEM((CH, 3) if emit_sf else (8, 3), f32)
          for _ in range(2)],
        *[pltpu.SemaphoreType.DMA for _ in range(4)],
        *[pltpu.SemaphoreType.DMA for _ in range(2)],
        *[pltpu.SemaphoreType.DMA for _ in range(2)],
        *[pltpu.SemaphoreType.DMA for _ in range(2)],
    )

    @functools.partial(pl.kernel, out_type=tuple(outs), mesh=mesh,
                       scratch_types=scratch,
                       compiler_params=pltpu.CompilerParams(
                           needs_layout_passes=False,
                           use_tc_tiling_on_sc=False),
                       name=f"edge_i{int(integrate)}_sf{int(emit_sf)}")
    def call(S_in, F_in, masses, ce16, cf16, idx1v, idx2v, kv, ev, z16,
             *rest):
        if emit_sf:
            S_out, F_out, SF_out = rest[:3]
            scr = rest[3:]
        else:
            S_out, F_out = rest[:2]
            SF_out = None
            scr = rest[2:]
        (S_sh, F_sh, PS, PF0, PF1, PM, PSN, CEb, CFb) = scr[:9]
        scr = list(scr[9:])

        def take(n):
            out, scr[:n] = scr[:n], []
            return out

        I1 = take(4)
        I2 = take(4)
        KB = take(4)
        EB = take(4)
        P1 = take(2)
        P2 = take(2)
        Q1 = take(2)
        Q2 = take(2)
        SFB = take(2)
        psem = take(4)
        gsem = take(2)
        ssem = take(2)
        fsem = take(2)
        cid = lax.axis_index("c")
        sid = lax.axis_index("s")
        wid = cid * NSC + sid
        tb = sid * RPT

        pltpu.sync_copy(z16.at[pl.ds(tb, RPT)], F_sh.at[pl.ds(tb, RPT)])
        pltpu.sync_copy(ce16, CEb)
        pltpu.sync_copy(cf16, CFb)

        if not integrate:
            pltpu.sync_copy(S_in.at[pl.ds(tb, RPT)], S_sh.at[pl.ds(tb, RPT)])
        else:
            for k in range(RPT // PR):
                rows0 = tb + k * PR
                c1 = pltpu.async_copy(S_in.at[pl.ds(rows0, PR)], PS, psem[0])
                c2 = pltpu.async_copy(F_in.at[0, pl.ds(rows0, PR)], PF0,
                                      psem[1])
                c3 = pltpu.async_copy(F_in.at[1, pl.ds(rows0, PR)], PF1,
                                      psem[2])
                c4 = pltpu.async_copy(masses.at[pl.ds(rows0, PR)], PM,
                                      psem[3])
                c1.wait()
                c2.wait()
                c3.wait()
                c4.wait()

                @pl.loop(0, PR // L)
                def _grp(g):
                    rvec, x3, v3 = _integrate_group(g, PS, PF0, PF1, PM,
                                                    CEb, CFb)
                    for c in range(3):
                        plsc.store_scatter(PSN, [rvec, _col(c)], x3[c])
                        plsc.store_scatter(PSN, [rvec, _col(4 + c)], v3[c])

                pltpu.sync_copy(PSN, S_sh.at[pl.ds(rows0, PR)])

                @pl.when(cid == 0)
                def _():
                    pltpu.sync_copy(PSN, S_out.at[pl.ds(rows0, PR)])

        for b in range(2):
            pltpu.sync_copy(z16.at[pl.ds(0, CH)], Q1[b])
            pltpu.sync_copy(z16.at[pl.ds(0, CH)], Q2[b])

        plsc.subcore_barrier()

        def params_issue(ch, pb):
            base = wid * SPT_PAD + ch * CH
            pltpu.async_copy(idx1v.at[pl.ds(base, CH)], I1[pb], psem[pb])
            pltpu.async_copy(idx2v.at[pl.ds(base, CH)], I2[pb], psem[pb])
            pltpu.async_copy(kv.at[pl.ds(base, CH)], KB[pb], psem[pb])
            pltpu.async_copy(ev.at[pl.ds(base, CH)], EB[pb], psem[pb])

        def params_drain(pb):
            pltpu.make_async_copy(idx1v.at[pl.ds(0, CH)], I1[pb],
                                  psem[pb]).wait()
            pltpu.make_async_copy(idx2v.at[pl.ds(0, CH)], I2[pb],
                                  psem[pb]).wait()
            pltpu.make_async_copy(kv.at[pl.ds(0, CH)], KB[pb],
                                  psem[pb]).wait()
            pltpu.make_async_copy(ev.at[pl.ds(0, CH)], EB[pb],
                                  psem[pb]).wait()

        def scatter_drain(b):
            pltpu.make_async_copy(z16.at[pl.ds(0, CH)], Q1[b],
                                  ssem[b]).wait()
            pltpu.make_async_copy(z16.at[pl.ds(0, CH)], Q2[b],
                                  ssem[b]).wait()

        def sf_drain(b):
            pltpu.make_async_copy(SF_out.at[pl.ds(0, CH)], SFB[b],
                                  fsem[b]).wait()

        def compute(b, pb):
            @pl.loop(0, GPC // 2)
            def _grp(gh):
                for gb in range(2):
                    g = gh * 2 + gb
                    iot = _iota()
                    rvec = g * L + iot
                    x1 = [plsc.load_gather(P1[b], [rvec, _col(c)])
                          for c in range(3)]
                    v1 = [plsc.load_gather(P1[b], [rvec, _col(4 + c)])
                          for c in range(3)]
                    x2 = [plsc.load_gather(P2[b], [rvec, _col(c)])
                          for c in range(3)]
                    v2 = [plsc.load_gather(P2[b], [rvec, _col(4 + c)])
                          for c in range(3)]
                    d0 = x2[0] - x1[0]
                    d1 = x2[1] - x1[1]
                    d2 = x2[2] - x1[2]
                    dn2 = d0 * d0 + d1 * d1 + d2 * d2
                    r = _rsqrt(dn2)
                    krl = KB[pb][pl.ds(g * L, L)]
                    e = EB[pb][pl.ds(g * L, L)]
                    sr = krl - e * r
                    vr = ((v2[0] - v1[0]) * d0 + (v2[1] - v1[1]) * d1 +
                          (v2[2] - v1[2]) * d2)
                    ts = sr + DASHPOT * (vr * r) * r
                    fx, fy, fz = ts * d0, ts * d1, ts * d2
                    plsc.store_scatter(Q1[b], [rvec, _col(0)], fx)
                    plsc.store_scatter(Q1[b], [rvec, _col(1)], fy)
                    plsc.store_scatter(Q1[b], [rvec, _col(2)], fz)
                    plsc.store_scatter(Q2[b], [rvec, _col(0)], -fx)
                    plsc.store_scatter(Q2[b], [rvec, _col(1)], -fy)
                    plsc.store_scatter(Q2[b], [rvec, _col(2)], -fz)
                    if emit_sf:
                        plsc.store_scatter(SFB[b], [rvec, _col(0)], sr * d0)
                        plsc.store_scatter(SFB[b], [rvec, _col(1)], sr * d1)
                        plsc.store_scatter(SFB[b], [rvec, _col(2)], sr * d2)

        def slot(ch, b, pb, prefetch, first):
            params_drain(pb)
            g1 = pltpu.async_copy(S_sh.at[I1[pb]], P1[b], gsem[b])
            g2 = pltpu.async_copy(S_sh.at[I2[pb]], P2[b], gsem[b])
            if not first:
                scatter_drain(b)
                if emit_sf:
                    sf_drain(b)
            if prefetch:
                params_issue(ch + 2, (pb + 2) % 4)
            g1.wait()
            g2.wait()
            compute(b, pb)
            pltpu.async_copy(Q1[b], F_sh.at[I1[pb]], ssem[b], add=True)
            pltpu.async_copy(Q2[b], F_sh.at[I2[pb]], ssem[b], add=True)
            if emit_sf:
                base = wid * SPT_PAD + ch * CH
                pltpu.async_copy(SFB[b], SF_out.at[pl.ds(base, CH)], fsem[b])

        params_issue(0, 0)
        params_issue(1, 1)
        slot(0, 0, 0, True, True)
        slot(1, 1, 1, True, True)

        @pl.loop(0, (NCHUNK - 4) // 4)
        def _chunk4(it):
            ch0 = 2 + it * 4
            for u in range(4):
                slot(ch0 + u, u % 2, (2 + u) % 4, True, False)

        slot(NCHUNK - 2, 0, 2, False, False)
        slot(NCHUNK - 1, 1, 3, False, False)
        for b in range(2):
            scatter_drain(b)
            if emit_sf:
                sf_drain(b)

        plsc.subcore_barrier()
        pltpu.sync_copy(F_sh.at[pl.ds(tb, RPT)],
                        F_out.at[cid, pl.ds(tb, RPT)])

    return call


def _make_final(mesh):
    scratch = (
        pltpu.VMEM((PR, 8), f32),
        pltpu.VMEM((PR, 16), f32),
        pltpu.VMEM((PR, 16), f32),
        pltpu.VMEM((PR,), f32),
        pltpu.VMEM((PR, 3), f32),
        pltpu.VMEM((L,), f32),
        pltpu.VMEM((L,), f32),
    )

    @functools.partial(pl.kernel,
                       out_type=jax.ShapeDtypeStruct((NV_PAD, 3), f32),
                       mesh=mesh, scratch_types=scratch,
                       compiler_params=pltpu.CompilerParams(
                           needs_layout_passes=False,
                           use_tc_tiling_on_sc=False),
                       name="final_integrate")
    def call(S_in, F_in, masses, ce16, cf16, x_out,
             PS, PF0, PF1, PM, XB, CEb, CFb):
        cid = lax.axis_index("c")
        sid = lax.axis_index("s")
        wid = cid * NSC + sid
        tb = wid * RPT32
        pltpu.sync_copy(ce16, CEb)
        pltpu.sync_copy(cf16, CFb)
        for k in range(RPT32 // PR):
            rows0 = tb + k * PR
            pltpu.sync_copy(S_in.at[pl.ds(rows0, PR)], PS)
            pltpu.sync_copy(F_in.at[0, pl.ds(rows0, PR)], PF0)
            pltpu.sync_copy(F_in.at[1, pl.ds(rows0, PR)], PF1)
            pltpu.sync_copy(masses.at[pl.ds(rows0, PR)], PM)

            @pl.loop(0, PR // L)
            def _grp(g):
                rvec, x3, _ = _integrate_group(g, PS, PF0, PF1, PM, CEb, CFb)
                for c in range(3):
                    plsc.store_scatter(XB, [rvec, _col(c)], x3[c])

            pltpu.sync_copy(XB, x_out.at[pl.ds(rows0, PR)])

    return call


_CALLS = {}


def _get_calls():
    if not _CALLS:
        mesh = plsc.VectorSubcoreMesh(core_axis_name="c", subcore_axis_name="s",
                                      num_cores=NC, num_subcores=NSC)
        _CALLS["edge0"] = _make_edge(mesh, integrate=False, emit_sf=False)
        _CALLS["edge_mid"] = _make_edge(mesh, integrate=True, emit_sf=False)
        _CALLS["edge_sf"] = _make_edge(mesh, integrate=True, emit_sf=True)
        _CALLS["final"] = _make_final(mesh)
    return _CALLS


def kernel(init_vertices, init_springs, init_rest_lengths, init_masses,
           spring_Y, collide_elas, collide_fric):
    S0 = jnp.zeros((NV_PAD, 8), f32).at[:NV, 0:3].set(init_vertices)
    masses_p = jnp.ones((NV_PAD,), f32).at[:NV].set(init_masses)
    idx1 = init_springs[:, 0]
    idx2 = init_springs[:, 1]

    def pad_springs(a, fill):
        a2 = a.reshape(NT, SPT)
        padblock = jnp.full((NT, SPT_PAD - SPT), fill, a.dtype)
        return jnp.concatenate([a2, padblock], axis=1).reshape(-1)

    i1p = pad_springs(idx1, NV)
    i2p = pad_springs(idx2, NV)
    e_spring = jnp.exp(spring_Y)
    kp = pad_springs(e_spring / init_rest_lengths, 0.0)
    ep = pad_springs(e_spring, 0.0)
    z16 = jnp.zeros((NV_PAD, 16), f32)
    ce16 = jnp.full((L,), collide_elas, f32)
    cf16 = jnp.full((L,), collide_fric, f32)
    fdummy = jnp.zeros((NC, NV_PAD, 16), f32)

    calls = _get_calls()
    common = (masses_p, ce16, cf16, i1p, i2p, kp, ep, z16)
    _, F0 = calls["edge0"](S0, fdummy, *common)
    S1, F1 = calls["edge_mid"](S0, F0, *common)
    S2, F2 = calls["edge_mid"](S1, F1, *common)
    S3, F3, SFp = calls["edge_sf"](S2, F2, *common)
    xp = calls["final"](S3, F3, masses_p, ce16, cf16)

    x = xp[:NV]
    sf = SFp.reshape(NT, SPT_PAD, 3)[:, :SPT].reshape(NS, 3)
    return (x, init_springs, init_rest_lengths, sf)
